# baseline (speedup 1.0000x reference)
import jax
import jax.numpy as jnp
from jax.experimental import pallas as pl
from jax.experimental.pallas import tpu as pltpu

_K = 8

_DIMS = (3, 3, 4, 3, 45, 1)


def _compute_chunk(v, camT, xyz, sc, rot, sh0, shr, op):
    a = jnp.concatenate([xyz, sc, rot, sh0, shr, op], axis=1)
    at = jnp.transpose(a, (1, 0))
    xyzT = at[0:3]
    scT = at[3:6]
    rotT = at[6:10]
    sh0T = at[10:13]
    shrT = at[13:58]
    opT = at[58:59]

    cscT = jnp.exp(scT * v)
    q0 = rotT[0:1]; q1 = rotT[1:2]; q2 = rotT[2:3]; q3 = rotT[3:4]
    qn = jnp.sqrt(q0 * q0 + q1 * q1 + q2 * q2 + q3 * q3)
    crtT = rotT * (v / (qn + 1e-8))
    copT = 1.0 / (1.0 + jnp.exp(-(opT * v)))

    d = xyzT - camT
    dx = d[0:1]; dy = d[1:2]; dz = d[2:3]
    dn = jnp.sqrt(dx * dx + dy * dy + dz * dz)
    inv = 1.0 / (dn + 1e-8)
    x = dx * inv; y = dy * inv; z = dz * inv

    res = 0.28209479177387814 * sh0T
    res = res - (0.4886025119029199 * y) * shrT[0:3]
    res = res + (0.4886025119029199 * z) * shrT[3:6]
    res = res - (0.4886025119029199 * x) * shrT[6:9]
    xx = x * x; yy = y * y; zz = z * z
    xy = x * y; yz = y * z; xz = x * z
    res = res + (1.0925484305920792 * xy) * shrT[9:12]
    res = res - (1.0925484305920792 * yz) * shrT[12:15]
    res = res + (0.31539156525252005 * (2.0 * zz - xx - yy)) * shrT[15:18]
    res = res - (1.0925484305920792 * xz) * shrT[18:21]
    res = res + (0.5462742152960396 * (xx - yy)) * shrT[21:24]
    res = res - (0.5900435899266435 * y * (3.0 * xx - yy)) * shrT[24:27]
    res = res + (2.890611442640554 * xy * z) * shrT[27:30]
    res = res - (0.4570457994644658 * y * (4.0 * zz - xx - yy)) * shrT[30:33]
    res = res + (0.3731763325901154 * z * (2.0 * zz - 3.0 * xx - 3.0 * yy)) * shrT[33:36]
    res = res - (0.4570457994644658 * x * (4.0 * zz - xx - yy)) * shrT[36:39]
    res = res + (1.445305721320277 * z * (xx - yy)) * shrT[39:42]
    res = res - (0.5900435899266435 * x * (xx - yy - zz)) * shrT[42:45]
    colT = jnp.maximum(res * v + 0.5, 0.0)

    ot = jnp.concatenate([xyzT * v, cscT, crtT, colT, copT], axis=0)
    return jnp.transpose(ot, (1, 0))


def _body(order_ref, cnt_ref, camT_ref, *refs):
    k = (len(refs) - 5) // 6
    ins = refs[:6 * k]
    cx_ref, csc_ref, crt_ref, col_ref, cop_ref = refs[6 * k:]
    i = pl.program_id(0)
    camT = camT_ref[...]
    for j in range(k):
        xyz_ref, sc_ref, rot_ref, sh0_ref, shr_ref, op_ref = ins[6 * j:6 * j + 6]
        v = jnp.where(k * i + j < cnt_ref[0], 1.0, 0.0).astype(jnp.float32)
        o = _compute_chunk(v, camT, xyz_ref[0], sc_ref[0], rot_ref[0],
                           sh0_ref[0], shr_ref[0], op_ref[0])
        cx_ref[j] = o[:, 0:3]
        csc_ref[j] = o[:, 3:6]
        crt_ref[j] = o[:, 6:10]
        col_ref[j] = o[:, 10:13]
        cop_ref[j] = o[:, 13:14]


def _gather_compute(order, cnt, camT, xyz_c, sc_c, rot_c, sh0_c, shr_c, op_c):
    c, cs, _ = xyz_c.shape
    k = _K if c % _K == 0 else 1
    arrays = (xyz_c, sc_c, rot_c, sh0_c, shr_c, op_c)

    in_specs = [pl.BlockSpec((3, cs), lambda i, o, n: (0, 0))]
    ins = [camT]
    for j in range(k):
        idx_map = (lambda jj: lambda i, o, n: (o[k * i + jj], 0, 0))(j)
        for a, d in zip(arrays, _DIMS):
            in_specs.append(pl.BlockSpec((1, cs, d), idx_map))
            ins.append(a)
    out_specs = [pl.BlockSpec((k, cs, d), lambda i, o, n: (i, 0, 0))
                 for d in (3, 3, 4, 3, 1)]
    out_shapes = [jax.ShapeDtypeStruct((c, cs, d), jnp.float32)
                  for d in (3, 3, 4, 3, 1)]

    grid_spec = pltpu.PrefetchScalarGridSpec(
        num_scalar_prefetch=2,
        grid=(c // k,),
        in_specs=in_specs,
        out_specs=out_specs,
    )
    return pl.pallas_call(
        _body,
        grid_spec=grid_spec,
        out_shape=out_shapes,
        compiler_params=pltpu.CompilerParams(
            dimension_semantics=("arbitrary",)),
    )(order, cnt, camT, *ins[1:])


def kernel(view_matrix, frustumplane, idx_tensor, feedback_visible_chunks_num,
           xyz, scale, rot, sh_0, sh_rest, opacity, cluster_origin,
           cluster_extend):
    c = cluster_origin.shape[0]
    n = xyz.shape[0]
    cs = n // c

    nrm = frustumplane[:, :3]
    dpl = frustumplane[:, 3]
    dist = (cluster_origin @ nrm.T + cluster_extend @ jnp.abs(nrm).T
            + dpl[None, :])
    mask = jnp.all(dist >= 0.0, axis=1)
    cnt = jnp.sum(mask.astype(jnp.int32))
    keys = jnp.where(mask, 0, 1) * c + jnp.arange(c)
    order = jnp.argsort(keys).astype(jnp.int32)
    visible_chunkid = jnp.take(idx_tensor, order)

    cam = view_matrix[3, :3]
    camT = jnp.broadcast_to(cam[:, None], (3, cs))

    cx, csc, crt, col, cop = _gather_compute(
        order, cnt.reshape(1), camT,
        xyz.reshape(c, cs, 3), scale.reshape(c, cs, 3),
        rot.reshape(c, cs, 4), sh_0.reshape(c, cs, 3),
        sh_rest.reshape(c, cs, 45), opacity.reshape(c, cs, 1))

    valid_length = cnt * cs
    return (visible_chunkid, cnt, valid_length,
            cx.reshape(n, 3), csc.reshape(n, 3), crt.reshape(n, 4),
            col.reshape(n, 3), cop.reshape(n, 1))


# speedup vs baseline: 10.1628x; 10.1628x over previous
import jax
import jax.numpy as jnp
from jax.experimental import pallas as pl
from jax.experimental.pallas import tpu as pltpu

_K = 8


def _compute_chunk(v, camT, xyzT, scT, rotT, sh0T, shrT, opT):
    cscT = jnp.exp(scT * v)

    q0 = rotT[0:1]; q1 = rotT[1:2]; q2 = rotT[2:3]; q3 = rotT[3:4]
    qn = jnp.sqrt(q0 * q0 + q1 * q1 + q2 * q2 + q3 * q3)
    crtT = rotT * (v / (qn + 1e-8))

    copT = 1.0 / (1.0 + jnp.exp(-(opT * v)))

    d = xyzT - camT
    dx = d[0:1]; dy = d[1:2]; dz = d[2:3]
    dn = jnp.sqrt(dx * dx + dy * dy + dz * dz)
    inv = 1.0 / (dn + 1e-8)
    x = dx * inv; y = dy * inv; z = dz * inv

    res = 0.28209479177387814 * sh0T
    res = res - (0.4886025119029199 * y) * shrT[:, 0]
    res = res + (0.4886025119029199 * z) * shrT[:, 1]
    res = res - (0.4886025119029199 * x) * shrT[:, 2]
    xx = x * x; yy = y * y; zz = z * z
    xy = x * y; yz = y * z; xz = x * z
    res = res + (1.0925484305920792 * xy) * shrT[:, 3]
    res = res - (1.0925484305920792 * yz) * shrT[:, 4]
    res = res + (0.31539156525252005 * (2.0 * zz - xx - yy)) * shrT[:, 5]
    res = res - (1.0925484305920792 * xz) * shrT[:, 6]
    res = res + (0.5462742152960396 * (xx - yy)) * shrT[:, 7]
    res = res - (0.5900435899266435 * y * (3.0 * xx - yy)) * shrT[:, 8]
    res = res + (2.890611442640554 * xy * z) * shrT[:, 9]
    res = res - (0.4570457994644658 * y * (4.0 * zz - xx - yy)) * shrT[:, 10]
    res = res + (0.3731763325901154 * z * (2.0 * zz - 3.0 * xx - 3.0 * yy)) * shrT[:, 11]
    res = res - (0.4570457994644658 * x * (4.0 * zz - xx - yy)) * shrT[:, 12]
    res = res + (1.445305721320277 * z * (xx - yy)) * shrT[:, 13]
    res = res - (0.5900435899266435 * x * (xx - yy - zz)) * shrT[:, 14]
    colT = jnp.maximum(res * v + 0.5, 0.0)

    return xyzT * v, cscT, crtT, colT, copT


def _body(order_ref, cnt_ref, camT_ref, *refs):
    k = (len(refs) - 5) // 6
    ins = refs[:6 * k]
    cx_ref, csc_ref, crt_ref, col_ref, cop_ref = refs[6 * k:]
    i = pl.program_id(0)
    camT = camT_ref[...]
    for j in range(k):
        xyz_ref, sc_ref, rot_ref, sh0_ref, shr_ref, op_ref = ins[6 * j:6 * j + 6]
        v = jnp.where(k * i + j < cnt_ref[0], 1.0, 0.0).astype(jnp.float32)
        cs = camT.shape[-1]
        cx, csc, crt, col, cop = _compute_chunk(
            v, camT, xyz_ref[0], sc_ref[0], rot_ref[0],
            jnp.reshape(sh0_ref[...], (3, cs)), shr_ref[...], op_ref[0])
        cx_ref[j] = cx
        csc_ref[j] = csc
        crt_ref[j] = crt
        col_ref[j] = col
        cop_ref[j] = cop


def _gather_compute(order, cnt, camT, xyz_v, sc_v, rot_v, sh0_v, shr_v, op_v):
    c, _, cs = xyz_v.shape
    k = _K if c % _K == 0 else 1

    chunk_specs = []
    ins = []
    for j in range(k):
        idx3 = (lambda jj: lambda i, o, n: (o[k * i + jj], 0, 0))(j)
        idx_shr = (lambda jj: lambda i, o, n: (0, 0, o[k * i + jj]))(j)
        chunk_specs += [
            pl.BlockSpec((1, 3, cs), idx3),
            pl.BlockSpec((1, 3, cs), idx3),
            pl.BlockSpec((1, 4, cs), idx3),
            pl.BlockSpec((3, 1, cs), idx_shr),
            pl.BlockSpec((3, 15, cs), idx_shr),
            pl.BlockSpec((1, 1, cs), idx3),
        ]
        ins += [xyz_v, sc_v, rot_v, sh0_v, shr_v, op_v]

    in_specs = [pl.BlockSpec((3, cs), lambda i, o, n: (0, 0))] + chunk_specs
    out_specs = [pl.BlockSpec((k, d, cs), lambda i, o, n: (i, 0, 0))
                 for d in (3, 3, 4, 3, 1)]
    out_shapes = [jax.ShapeDtypeStruct((c, d, cs), jnp.float32)
                  for d in (3, 3, 4, 3, 1)]

    grid_spec = pltpu.PrefetchScalarGridSpec(
        num_scalar_prefetch=2,
        grid=(c // k,),
        in_specs=in_specs,
        out_specs=out_specs,
    )
    return pl.pallas_call(
        _body,
        grid_spec=grid_spec,
        out_shape=out_shapes,
        compiler_params=pltpu.CompilerParams(
            dimension_semantics=("arbitrary",)),
    )(order, cnt, camT, *ins)


def kernel(view_matrix, frustumplane, idx_tensor, feedback_visible_chunks_num,
           xyz, scale, rot, sh_0, sh_rest, opacity, cluster_origin,
           cluster_extend):
    c = cluster_origin.shape[0]
    n = xyz.shape[0]
    cs = n // c

    nrm = frustumplane[:, :3]
    dpl = frustumplane[:, 3]
    dist = (cluster_origin @ nrm.T + cluster_extend @ jnp.abs(nrm).T
            + dpl[None, :])
    mask = jnp.all(dist >= 0.0, axis=1)
    cnt = jnp.sum(mask.astype(jnp.int32))
    keys = jnp.where(mask, 0, 1) * c + jnp.arange(c)
    order = jnp.argsort(keys).astype(jnp.int32)
    visible_chunkid = jnp.take(idx_tensor, order)

    cam = view_matrix[3, :3]
    camT = jnp.broadcast_to(cam[:, None], (3, cs))

    xyz_v = xyz.reshape(c, cs, 3).transpose(0, 2, 1)
    sc_v = scale.reshape(c, cs, 3).transpose(0, 2, 1)
    rot_v = rot.reshape(c, cs, 4).transpose(0, 2, 1)
    op_v = opacity.reshape(c, cs, 1).transpose(0, 2, 1)
    sh0_v = sh_0.transpose(2, 1, 0)
    shr_v = sh_rest.transpose(2, 1, 0)

    cx, csc, crt, col, cop = _gather_compute(
        order, cnt.reshape(1), camT, xyz_v, sc_v, rot_v, sh0_v, shr_v, op_v)

    valid_length = cnt * cs
    return (visible_chunkid, cnt, valid_length,
            cx.transpose(0, 2, 1).reshape(n, 3),
            csc.transpose(0, 2, 1).reshape(n, 3),
            crt.transpose(0, 2, 1).reshape(n, 4),
            col.transpose(0, 2, 1).reshape(n, 3),
            cop.transpose(0, 2, 1).reshape(n, 1))
